# Initial kernel scaffold; baseline (speedup 1.0000x reference)
import jax
import jax.numpy as jnp
from jax.experimental import pallas as pl


def kernel(x, triples):
    raise NotImplementedError("write your pallas kernel here")


# speedup vs baseline: 12.1312x; 12.1312x over previous
import functools

import jax
import jax.numpy as jnp
from jax import lax
from jax.experimental import pallas as pl
from jax.experimental.pallas import tpu as pltpu
from jax.experimental.pallas import tpu_sc as plsc


def _build_sc_call(N, D, NCH):
    Dh = D // 2
    NS = 16
    CH = 128
    SLAB = 640
    R_ACC = NS * SLAB
    ZR = 128
    NCHK = 80

    mesh = plsc.VectorSubcoreMesh(core_axis_name="c", subcore_axis_name="s")

    @functools.partial(
        pl.kernel,
        mesh=mesh,
        compiler_params=pltpu.CompilerParams(use_tc_tiling_on_sc=False),
        out_type=[jax.ShapeDtypeStruct((N, Dh), jnp.float32),
                  jax.ShapeDtypeStruct((N, Dh), jnp.float32)],
        scratch_types=[
            pltpu.VMEM((NCH, CH), jnp.int32),
            pltpu.VMEM((NCH, CH), jnp.int32),
            pltpu.VMEM((CH, Dh), jnp.float32),
            pltpu.VMEM((CH,), jnp.float32),
            pltpu.VMEM((ZR, Dh), jnp.float32),
            pltpu.VMEM((SLAB,), jnp.float32),
            pltpu.VMEM((NCHK, Dh), jnp.float32),
            pltpu.VMEM((SLAB,), jnp.float32),
            pltpu.VMEM((SLAB,), jnp.float32),
            pltpu.VMEM_SHARED((R_ACC, Dh), jnp.float32),
            pltpu.VMEM_SHARED((R_ACC,), jnp.float32),
            pltpu.SemaphoreType.DMA,
        ],
    )
    def sc_fn(xl_h, xr_h, s_h, r_h, outl_h, outr_h,
              sidx, ridx, gbuf, ones, zbuf, zvec, pslab, dslab, islab,
              acc, dacc, sem):
        c = lax.axis_index("c")
        s = lax.axis_index("s")
        base = s * SLAB

        def fill_zb(i, carry):
            for k in range(Dh // 16):
                zbuf[i, pl.ds(k * 16, 16)] = jnp.zeros((16,), jnp.float32)
            return carry
        lax.fori_loop(0, ZR, fill_zb, 0)

        def fill_zv(i, carry):
            zvec[pl.ds(i * 16, 16)] = jnp.zeros((16,), jnp.float32)
            return carry
        lax.fori_loop(0, SLAB // 16, fill_zv, 0)

        def fill_on(i, carry):
            ones[pl.ds(i * 16, 16)] = jnp.ones((16,), jnp.float32)
            return carry
        lax.fori_loop(0, CH // 16, fill_on, 0)

        for b in range(SLAB // ZR):
            pltpu.sync_copy(zbuf, acc.at[pl.ds(base + b * ZR, ZR)])
        pltpu.sync_copy(zvec, dacc.at[pl.ds(base, SLAB)])

        pltpu.sync_copy(s_h.at[s], sidx)
        pltpu.sync_copy(r_h.at[s], ridx)

        plsc.subcore_barrier()

        def accumulate(xh):
            def chunk(j, carry):
                pltpu.async_copy(xh.at[sidx.at[j]], gbuf, sem).wait()
                pltpu.sync_copy(gbuf, acc.at[ridx.at[j]], add=True)
                pltpu.sync_copy(ones, dacc.at[ridx.at[j]], add=True)
                return carry
            lax.fori_loop(0, NCH, chunk, 0)

        @pl.when(c == 0)
        def _():
            accumulate(xl_h)

        @pl.when(c == 1)
        def _():
            accumulate(xr_h)

        plsc.subcore_barrier()

        pltpu.sync_copy(dacc.at[pl.ds(base, SLAB)], dslab)

        def inv_f(i, carry):
            d = dslab[pl.ds(i * 16, 16)]
            islab[pl.ds(i * 16, 16)] = 1.0 / jnp.maximum(d, 1.0)
            return carry
        lax.fori_loop(0, SLAB // 16, inv_f, 0)

        def writeout(outh):
            def norm_chunk(k, carry):
                row0 = base + k * NCHK

                @pl.when(row0 + NCHK <= N)
                def _():
                    pltpu.sync_copy(acc.at[pl.ds(row0, NCHK)], pslab)
                    for g in range(NCHK // 16):
                        ivv = islab[pl.ds(k * NCHK + g * 16, 16)]
                        for t in range(16):
                            iv = ivv[t]
                            r = g * 16 + t
                            for kk in range(Dh // 16):
                                pslab[r, pl.ds(kk * 16, 16)] = (
                                    pslab[r, pl.ds(kk * 16, 16)] * iv)
                    pltpu.sync_copy(pslab, outh.at[pl.ds(row0, NCHK)])
                return carry
            lax.fori_loop(0, SLAB // NCHK, norm_chunk, 0)

        @pl.when(c == 0)
        def _():
            writeout(outl_h)

        @pl.when(c == 1)
        def _():
            writeout(outr_h)

    return sc_fn


def kernel(x, triples):
    N, D = x.shape
    NS, CH = 16, 128
    t0 = triples[:, 0]
    t2 = triples[:, 2]
    senders = jnp.concatenate([t0, t2])
    receivers = jnp.concatenate([t2, t0])
    E = senders.shape[0]

    nch = -(-E // (NS * CH))
    nch += nch % 2
    epad = NS * nch * CH
    pad = epad - E
    senders_p = jnp.concatenate(
        [senders, jnp.zeros((pad,), jnp.int32)]).reshape(NS, nch, CH)
    receivers_p = jnp.concatenate(
        [receivers, jnp.full((pad,), N, jnp.int32)]).reshape(NS, nch, CH)

    xl = x[:, : D // 2]
    xr = x[:, D // 2:]

    sc_fn = _build_sc_call(N, D, nch)
    out_l, out_r = sc_fn(xl, xr, senders_p, receivers_p)
    return jnp.concatenate([out_l, out_r], axis=1)


# speedup vs baseline: 13.8646x; 1.1429x over previous
import functools

import jax
import jax.numpy as jnp
from jax import lax
from jax.experimental import pallas as pl
from jax.experimental.pallas import tpu as pltpu
from jax.experimental.pallas import tpu_sc as plsc

NBUF = 2


def _build_sc_call(N, D, NCH):
    Dh = D // 2
    NS = 16
    CH = 128
    SLAB = 640
    R_ACC = NS * SLAB
    NCHK = 80

    mesh = plsc.VectorSubcoreMesh(core_axis_name="c", subcore_axis_name="s")

    @functools.partial(
        pl.kernel,
        mesh=mesh,
        compiler_params=pltpu.CompilerParams(use_tc_tiling_on_sc=False),
        out_type=[jax.ShapeDtypeStruct((N, Dh), jnp.float32),
                  jax.ShapeDtypeStruct((N, Dh), jnp.float32)],
        scratch_types=[
            pltpu.VMEM((NCH, CH), jnp.int32),
            pltpu.VMEM((NCH, CH), jnp.int32),
            pltpu.VMEM((NBUF, CH, Dh), jnp.float32),
            pltpu.VMEM((CH,), jnp.float32),
            pltpu.VMEM((SLAB,), jnp.float32),
            pltpu.VMEM((NCHK, Dh), jnp.float32),
            pltpu.VMEM((SLAB,), jnp.float32),
            pltpu.VMEM((SLAB,), jnp.float32),
            pltpu.VMEM_SHARED((R_ACC, Dh), jnp.float32),
            pltpu.VMEM_SHARED((R_ACC,), jnp.float32),
        ] + [pltpu.SemaphoreType.DMA] * NBUF,
    )
    def sc_fn(xl_h, xr_h, s_h, r_h, outl_h, outr_h,
              sidx, ridx, gbuf, ones, zvec, pslab, dslab, islab,
              acc, dacc, *sems):
        c = lax.axis_index("c")
        s = lax.axis_index("s")
        base = s * SLAB

        def fill_ps(i, carry):
            for k in range(Dh // 16):
                pslab[i, pl.ds(k * 16, 16)] = jnp.zeros((16,), jnp.float32)
            return carry
        lax.fori_loop(0, NCHK, fill_ps, 0)

        def fill_zv(i, carry):
            zvec[pl.ds(i * 16, 16)] = jnp.zeros((16,), jnp.float32)
            return carry
        lax.fori_loop(0, SLAB // 16, fill_zv, 0)

        def fill_on(i, carry):
            ones[pl.ds(i * 16, 16)] = jnp.ones((16,), jnp.float32)
            return carry
        lax.fori_loop(0, CH // 16, fill_on, 0)

        for b in range(SLAB // NCHK):
            pltpu.sync_copy(pslab, acc.at[pl.ds(base + b * NCHK, NCHK)])
        pltpu.sync_copy(zvec, dacc.at[pl.ds(base, SLAB)])

        pltpu.sync_copy(s_h.at[s], sidx)
        pltpu.sync_copy(r_h.at[s], ridx)

        plsc.subcore_barrier()

        def accumulate(xh):
            def rnd(i, carry):
                g = i * NBUF
                handles = [
                    pltpu.async_copy(xh.at[sidx.at[g + b]], gbuf.at[b], sems[b])
                    for b in range(NBUF)
                ]
                for b in range(NBUF):
                    handles[b].wait()
                    pltpu.sync_copy(gbuf.at[b], acc.at[ridx.at[g + b]],
                                    add=True)
                    pltpu.sync_copy(ones, dacc.at[ridx.at[g + b]], add=True)
                return carry
            lax.fori_loop(0, NCH // NBUF, rnd, 0)

        @pl.when(c == 0)
        def _():
            accumulate(xl_h)

        @pl.when(c == 1)
        def _():
            accumulate(xr_h)

        plsc.subcore_barrier()

        pltpu.sync_copy(dacc.at[pl.ds(base, SLAB)], dslab)

        def inv_f(i, carry):
            d = dslab[pl.ds(i * 16, 16)]
            islab[pl.ds(i * 16, 16)] = 1.0 / jnp.maximum(d, 1.0)
            return carry
        lax.fori_loop(0, SLAB // 16, inv_f, 0)

        def writeout(outh):
            def norm_chunk(k, carry):
                row0 = base + k * NCHK

                @pl.when(row0 + NCHK <= N)
                def _():
                    pltpu.sync_copy(acc.at[pl.ds(row0, NCHK)], pslab)
                    for g in range(NCHK // 16):
                        ivv = islab[pl.ds(k * NCHK + g * 16, 16)]
                        for t in range(16):
                            iv = ivv[t]
                            r = g * 16 + t
                            for kk in range(Dh // 16):
                                pslab[r, pl.ds(kk * 16, 16)] = (
                                    pslab[r, pl.ds(kk * 16, 16)] * iv)
                    pltpu.sync_copy(pslab, outh.at[pl.ds(row0, NCHK)])
                return carry
            lax.fori_loop(0, SLAB // NCHK, norm_chunk, 0)

        @pl.when(c == 0)
        def _():
            writeout(outl_h)

        @pl.when(c == 1)
        def _():
            writeout(outr_h)

    return sc_fn


def kernel(x, triples):
    N, D = x.shape
    NS, CH = 16, 128
    t0 = triples[:, 0]
    t2 = triples[:, 2]
    senders = jnp.concatenate([t0, t2])
    receivers = jnp.concatenate([t2, t0])
    E = senders.shape[0]

    nch = -(-E // (NS * CH))
    nch += -nch % NBUF
    epad = NS * nch * CH
    pad = epad - E
    senders_p = jnp.concatenate(
        [senders, jnp.zeros((pad,), jnp.int32)]).reshape(NS, nch, CH)
    receivers_p = jnp.concatenate(
        [receivers, jnp.full((pad,), N, jnp.int32)]).reshape(NS, nch, CH)

    xl = x[:, : D // 2]
    xr = x[:, D // 2:]

    sc_fn = _build_sc_call(N, D, nch)
    out_l, out_r = sc_fn(xl, xr, senders_p, receivers_p)
    return jnp.concatenate([out_l, out_r], axis=1)


# speedup vs baseline: 14.5273x; 1.0478x over previous
import functools

import jax
import jax.numpy as jnp
from jax import lax
from jax.experimental import pallas as pl
from jax.experimental.pallas import tpu as pltpu
from jax.experimental.pallas import tpu_sc as plsc

NBUF = 2


def _build_sc_call(N, D, NCH):
    Dh = D // 2
    NS = 16
    CH = 128
    SLAB = 640
    R_ACC = NS * SLAB
    NCHK = 80
    HR = R_ACC // CH

    mesh = plsc.VectorSubcoreMesh(core_axis_name="c", subcore_axis_name="s")

    @functools.partial(
        pl.kernel,
        mesh=mesh,
        compiler_params=pltpu.CompilerParams(use_tc_tiling_on_sc=False,
                                             needs_layout_passes=False),
        out_type=[jax.ShapeDtypeStruct((N, Dh), jnp.float32),
                  jax.ShapeDtypeStruct((N, Dh), jnp.float32)],
        scratch_types=[
            pltpu.VMEM((NCH, CH), jnp.int32),
            pltpu.VMEM((NCH, CH), jnp.int32),
            pltpu.VMEM((NBUF, CH, Dh), jnp.float32),
            pltpu.VMEM((HR, CH), jnp.float32),
            pltpu.VMEM((HR,), jnp.int32),
            pltpu.VMEM((NCHK, Dh), jnp.float32),
            pltpu.VMEM((HR // NS, CH), jnp.float32),
            pltpu.VMEM((HR // NS, CH), jnp.float32),
            pltpu.VMEM_SHARED((R_ACC, Dh), jnp.float32),
            pltpu.VMEM_SHARED((HR, CH), jnp.float32),
        ] + [pltpu.SemaphoreType.DMA] * NBUF,
    )
    def sc_fn(xl_h, xr_h, s_h, r_h, outl_h, outr_h,
              sidx, ridx, gbuf, hist, hrows, pslab, dslab, islab,
              acc, dacc, *sems):
        c = lax.axis_index("c")
        s = lax.axis_index("s")
        base = s * SLAB

        def fill_ps(i, carry):
            for k in range(Dh // 16):
                pslab[i, pl.ds(k * 16, 16)] = jnp.zeros((16,), jnp.float32)
            return carry
        lax.fori_loop(0, NCHK, fill_ps, 0)

        def fill_hist(i, carry):
            for k in range(CH // 16):
                hist[i, pl.ds(k * 16, 16)] = jnp.zeros((16,), jnp.float32)
            return carry
        lax.fori_loop(0, HR, fill_hist, 0)

        for g in range(HR // 16):
            hrows[pl.ds(g * 16, 16)] = lax.iota(jnp.int32, 16) + (g * 16)

        for b in range(SLAB // NCHK):
            pltpu.sync_copy(pslab, acc.at[pl.ds(base + b * NCHK, NCHK)])

        @pl.when(s == 0)
        def _():
            pltpu.sync_copy(hist, dacc)

        pltpu.sync_copy(s_h.at[s], sidx)
        pltpu.sync_copy(r_h.at[s], ridx)

        plsc.subcore_barrier()

        ones16 = jnp.ones((16,), jnp.float32)

        def accumulate(xh):
            def rnd(i, carry):
                g = i * NBUF
                handles = [
                    pltpu.async_copy(xh.at[sidx.at[g + b]], gbuf.at[b], sems[b])
                    for b in range(NBUF)
                ]
                for b in range(NBUF):
                    for k in range(CH // 16):
                        rv = ridx[g + b, pl.ds(k * 16, 16)]
                        rrow = lax.shift_right_logical(rv, 7)
                        rcol = lax.bitwise_and(rv, 127)
                        plsc.addupdate_scatter(hist, [rrow, rcol], ones16)
                for b in range(NBUF):
                    handles[b].wait()
                    pltpu.sync_copy(gbuf.at[b], acc.at[ridx.at[g + b]],
                                    add=True)
                return carry
            lax.fori_loop(0, NCH // NBUF, rnd, 0)

        @pl.when(c == 0)
        def _():
            accumulate(xl_h)

        @pl.when(c == 1)
        def _():
            accumulate(xr_h)

        pltpu.sync_copy(hist, dacc.at[hrows], add=True)

        plsc.subcore_barrier()

        pltpu.sync_copy(dacc.at[pl.ds(s * (HR // NS), HR // NS)], dslab)

        def inv_r(r, carry):
            for k in range(CH // 16):
                d = dslab[r, pl.ds(k * 16, 16)]
                islab[r, pl.ds(k * 16, 16)] = 1.0 / jnp.maximum(d, 1.0)
            return carry
        lax.fori_loop(0, HR // NS, inv_r, 0)

        def writeout(outh):
            def norm_chunk(k, carry):
                row0 = base + k * NCHK

                @pl.when(row0 + NCHK <= N)
                def _():
                    pltpu.sync_copy(acc.at[pl.ds(row0, NCHK)], pslab)
                    for g in range(NCHK // 16):
                        e0 = k * NCHK + g * 16
                        erow = e0 // CH
                        ecol = e0 % CH
                        ivv = islab[erow, pl.ds(ecol, 16)]
                        for t in range(16):
                            iv = ivv[t]
                            r = g * 16 + t
                            for kk in range(Dh // 16):
                                pslab[r, pl.ds(kk * 16, 16)] = (
                                    pslab[r, pl.ds(kk * 16, 16)] * iv)
                    pltpu.sync_copy(pslab, outh.at[pl.ds(row0, NCHK)])
                return carry
            lax.fori_loop(0, SLAB // NCHK, norm_chunk, 0)

        @pl.when(c == 0)
        def _():
            writeout(outl_h)

        @pl.when(c == 1)
        def _():
            writeout(outr_h)

    return sc_fn


def kernel(x, triples):
    N, D = x.shape
    NS, CH = 16, 128
    t0 = triples[:, 0]
    t2 = triples[:, 2]
    senders = jnp.concatenate([t0, t2])
    receivers = jnp.concatenate([t2, t0])
    E = senders.shape[0]

    nch = -(-E // (NS * CH))
    nch += -nch % NBUF
    epad = NS * nch * CH
    pad = epad - E
    senders_p = jnp.concatenate(
        [senders, jnp.zeros((pad,), jnp.int32)]).reshape(NS, nch, CH)
    receivers_p = jnp.concatenate(
        [receivers, jnp.full((pad,), N, jnp.int32)]).reshape(NS, nch, CH)

    xl = x[:, : D // 2]
    xr = x[:, D // 2:]

    sc_fn = _build_sc_call(N, D, nch)
    out_l, out_r = sc_fn(xl, xr, senders_p, receivers_p)
    return jnp.concatenate([out_l, out_r], axis=1)


# speedup vs baseline: 14.7630x; 1.0162x over previous
import functools

import jax
import jax.numpy as jnp
from jax import lax
from jax.experimental import pallas as pl
from jax.experimental.pallas import tpu as pltpu
from jax.experimental.pallas import tpu_sc as plsc

NBUF = 2


def _build_sc_call(N, D, NCH):
    Dh = D // 2
    NS = 16
    CH = 128
    SLAB = 640
    R_ACC = NS * SLAB
    NCHK = 80
    HR = R_ACC // CH

    mesh = plsc.VectorSubcoreMesh(core_axis_name="c", subcore_axis_name="s")

    @functools.partial(
        pl.kernel,
        mesh=mesh,
        compiler_params=pltpu.CompilerParams(use_tc_tiling_on_sc=False,
                                             needs_layout_passes=False),
        out_type=[jax.ShapeDtypeStruct((N, Dh), jnp.float32),
                  jax.ShapeDtypeStruct((N, Dh), jnp.float32)],
        scratch_types=[
            pltpu.VMEM((NCH, CH), jnp.int32),
            pltpu.VMEM((NCH, CH), jnp.int32),
            pltpu.VMEM((NBUF, CH, Dh), jnp.float32),
            pltpu.VMEM((HR, CH), jnp.float32),
            pltpu.VMEM((HR,), jnp.int32),
            pltpu.VMEM((NCHK, Dh), jnp.float32),
            pltpu.VMEM((HR // NS, CH), jnp.float32),
            pltpu.VMEM((HR // NS, CH), jnp.float32),
            pltpu.VMEM_SHARED((R_ACC, Dh), jnp.float32),
            pltpu.VMEM_SHARED((HR, CH), jnp.float32),
        ] + [pltpu.SemaphoreType.DMA] * (2 * NBUF),
    )
    def sc_fn(xl_h, xr_h, s_h, r_h, outl_h, outr_h,
              sidx, ridx, gbuf, hist, hrows, pslab, dslab, islab,
              acc, dacc, *sems):
        c = lax.axis_index("c")
        s = lax.axis_index("s")
        base = s * SLAB

        def fill_ps(i, carry):
            for k in range(Dh // 16):
                pslab[i, pl.ds(k * 16, 16)] = jnp.zeros((16,), jnp.float32)
            return carry
        lax.fori_loop(0, NCHK, fill_ps, 0)

        def fill_hist(i, carry):
            for k in range(CH // 16):
                hist[i, pl.ds(k * 16, 16)] = jnp.zeros((16,), jnp.float32)
            return carry
        lax.fori_loop(0, HR, fill_hist, 0)

        for g in range(HR // 16):
            hrows[pl.ds(g * 16, 16)] = lax.iota(jnp.int32, 16) + (g * 16)

        for b in range(SLAB // NCHK):
            pltpu.sync_copy(pslab, acc.at[pl.ds(base + b * NCHK, NCHK)])

        @pl.when(s == 0)
        def _():
            pltpu.sync_copy(hist, dacc)

        pltpu.sync_copy(s_h.at[s], sidx)
        pltpu.sync_copy(r_h.at[s], ridx)

        plsc.subcore_barrier()

        ones16 = jnp.ones((16,), jnp.float32)

        def accumulate(xh):
            def rnd(i, carry):
                g = i * NBUF
                handles = [
                    pltpu.async_copy(xh.at[sidx.at[g + b]], gbuf.at[b], sems[b])
                    for b in range(NBUF)
                ]
                for b in range(NBUF):
                    for k in range(CH // 16):
                        rv = ridx[g + b, pl.ds(k * 16, 16)]
                        rrow = lax.shift_right_logical(rv, 7)
                        rcol = lax.bitwise_and(rv, 127)
                        plsc.addupdate_scatter(hist, [rrow, rcol], ones16)
                shandles = []
                for b in range(NBUF):
                    handles[b].wait()
                    shandles.append(
                        pltpu.async_copy(gbuf.at[b], acc.at[ridx.at[g + b]],
                                         sems[NBUF + b], add=True))
                for h in shandles:
                    h.wait()
                return carry
            lax.fori_loop(0, NCH // NBUF, rnd, 0)

        @pl.when(c == 0)
        def _():
            accumulate(xl_h)

        @pl.when(c == 1)
        def _():
            accumulate(xr_h)

        pltpu.sync_copy(hist, dacc.at[hrows], add=True)

        plsc.subcore_barrier()

        pltpu.sync_copy(dacc.at[pl.ds(s * (HR // NS), HR // NS)], dslab)

        def inv_r(r, carry):
            for k in range(CH // 16):
                d = dslab[r, pl.ds(k * 16, 16)]
                islab[r, pl.ds(k * 16, 16)] = 1.0 / jnp.maximum(d, 1.0)
            return carry
        lax.fori_loop(0, HR // NS, inv_r, 0)

        def writeout(outh):
            def norm_chunk(k, carry):
                row0 = base + k * NCHK

                @pl.when(row0 + NCHK <= N)
                def _():
                    pltpu.sync_copy(acc.at[pl.ds(row0, NCHK)], pslab)
                    for g in range(NCHK // 16):
                        e0 = k * NCHK + g * 16
                        erow = e0 // CH
                        ecol = e0 % CH
                        ivv = islab[erow, pl.ds(ecol, 16)]
                        for t in range(16):
                            iv = ivv[t]
                            r = g * 16 + t
                            for kk in range(Dh // 16):
                                pslab[r, pl.ds(kk * 16, 16)] = (
                                    pslab[r, pl.ds(kk * 16, 16)] * iv)
                    pltpu.sync_copy(pslab, outh.at[pl.ds(row0, NCHK)])
                return carry
            lax.fori_loop(0, SLAB // NCHK, norm_chunk, 0)

        @pl.when(c == 0)
        def _():
            writeout(outl_h)

        @pl.when(c == 1)
        def _():
            writeout(outr_h)

    return sc_fn


def kernel(x, triples):
    N, D = x.shape
    NS, CH = 16, 128
    t0 = triples[:, 0]
    t2 = triples[:, 2]
    senders = jnp.concatenate([t0, t2])
    receivers = jnp.concatenate([t2, t0])
    E = senders.shape[0]

    nch = -(-E // (NS * CH))
    nch += -nch % NBUF
    epad = NS * nch * CH
    pad = epad - E
    senders_p = jnp.concatenate(
        [senders, jnp.zeros((pad,), jnp.int32)]).reshape(NS, nch, CH)
    receivers_p = jnp.concatenate(
        [receivers, jnp.full((pad,), N, jnp.int32)]).reshape(NS, nch, CH)

    xl = x[:, : D // 2]
    xr = x[:, D // 2:]

    sc_fn = _build_sc_call(N, D, nch)
    out_l, out_r = sc_fn(xl, xr, senders_p, receivers_p)
    return jnp.concatenate([out_l, out_r], axis=1)


# speedup vs baseline: 16.9993x; 1.1515x over previous
import functools

import jax
import jax.numpy as jnp
from jax import lax
from jax.experimental import pallas as pl
from jax.experimental.pallas import tpu as pltpu
from jax.experimental.pallas import tpu_sc as plsc

NBUF = 2
SB = 32


def _build_sc_call(N, D, NCH):
    Dh = D // 2
    NS = 16
    CH = 128
    SLAB = 640
    R_ACC = NS * SLAB
    NCHK = 80
    HR = R_ACC // CH
    XR = N // NS

    mesh = plsc.VectorSubcoreMesh(core_axis_name="c", subcore_axis_name="s")

    @functools.partial(
        pl.kernel,
        mesh=mesh,
        compiler_params=pltpu.CompilerParams(use_tc_tiling_on_sc=False,
                                             needs_layout_passes=False),
        out_type=[jax.ShapeDtypeStruct((N, Dh), jnp.float32),
                  jax.ShapeDtypeStruct((N, Dh), jnp.float32)],
        scratch_types=[
            pltpu.VMEM((SB, CH), jnp.int32),
            pltpu.VMEM((SB, CH), jnp.int32),
            pltpu.VMEM((NBUF, CH, Dh), jnp.float32),
            pltpu.VMEM((HR, CH), jnp.float32),
            pltpu.VMEM((HR,), jnp.int32),
            pltpu.VMEM((NCHK, Dh), jnp.float32),
            pltpu.VMEM((HR // NS, CH), jnp.float32),
            pltpu.VMEM((HR // NS, CH), jnp.float32),
            pltpu.VMEM_SHARED((R_ACC, Dh), jnp.float32),
            pltpu.VMEM_SHARED((HR, CH), jnp.float32),
            pltpu.VMEM_SHARED((N, Dh), jnp.float32),
        ] + [pltpu.SemaphoreType.DMA] * (2 * NBUF),
    )
    def sc_fn(xl_h, xr_h, s_h, r_h, outl_h, outr_h,
              sidx, ridx, gbuf, hist, hrows, pslab, dslab, islab,
              acc, dacc, xs, *sems):
        c = lax.axis_index("c")
        s = lax.axis_index("s")
        base = s * SLAB

        def fill_ps(i, carry):
            for k in range(Dh // 16):
                pslab[i, pl.ds(k * 16, 16)] = jnp.zeros((16,), jnp.float32)
            return carry
        lax.fori_loop(0, NCHK, fill_ps, 0)

        def fill_hist(i, carry):
            for k in range(CH // 16):
                hist[i, pl.ds(k * 16, 16)] = jnp.zeros((16,), jnp.float32)
            return carry
        lax.fori_loop(0, HR, fill_hist, 0)

        for g in range(HR // 16):
            hrows[pl.ds(g * 16, 16)] = lax.iota(jnp.int32, 16) + (g * 16)

        for b in range(SLAB // NCHK):
            pltpu.sync_copy(pslab, acc.at[pl.ds(base + b * NCHK, NCHK)])

        @pl.when(s == 0)
        def _():
            pltpu.sync_copy(hist, dacc)

        def stage_x(xh):
            pltpu.sync_copy(xh.at[pl.ds(s * XR, XR)], xs.at[pl.ds(s * XR, XR)])

        @pl.when(c == 0)
        def _():
            stage_x(xl_h)

        @pl.when(c == 1)
        def _():
            stage_x(xr_h)

        plsc.subcore_barrier()

        ones16 = jnp.ones((16,), jnp.float32)

        def stage(t, carry):
            pltpu.sync_copy(s_h.at[s, pl.ds(t * SB, SB)], sidx)
            pltpu.sync_copy(r_h.at[s, pl.ds(t * SB, SB)], ridx)

            def rnd(i, carry2):
                g = i * NBUF
                handles = [
                    pltpu.async_copy(xs.at[sidx.at[g + b]], gbuf.at[b],
                                     sems[b])
                    for b in range(NBUF)
                ]
                for b in range(NBUF):
                    for k in range(CH // 16):
                        rv = ridx[g + b, pl.ds(k * 16, 16)]
                        rrow = lax.shift_right_logical(rv, 7)
                        rcol = lax.bitwise_and(rv, 127)
                        plsc.addupdate_scatter(hist, [rrow, rcol], ones16)
                shandles = []
                for b in range(NBUF):
                    handles[b].wait()
                    shandles.append(
                        pltpu.async_copy(gbuf.at[b], acc.at[ridx.at[g + b]],
                                         sems[NBUF + b], add=True))
                for h in shandles:
                    h.wait()
                return carry2
            lax.fori_loop(0, SB // NBUF, rnd, 0)
            return carry
        lax.fori_loop(0, NCH // SB, stage, 0)

        pltpu.sync_copy(hist, dacc.at[hrows], add=True)

        plsc.subcore_barrier()

        pltpu.sync_copy(dacc.at[pl.ds(s * (HR // NS), HR // NS)], dslab)

        def inv_r(r, carry):
            for k in range(CH // 16):
                d = dslab[r, pl.ds(k * 16, 16)]
                islab[r, pl.ds(k * 16, 16)] = 1.0 / jnp.maximum(d, 1.0)
            return carry
        lax.fori_loop(0, HR // NS, inv_r, 0)

        def writeout(outh):
            def norm_chunk(k, carry):
                row0 = base + k * NCHK

                @pl.when(row0 + NCHK <= N)
                def _():
                    pltpu.sync_copy(acc.at[pl.ds(row0, NCHK)], pslab)
                    for g in range(NCHK // 16):
                        e0 = k * NCHK + g * 16
                        erow = e0 // CH
                        ecol = e0 % CH
                        ivv = islab[erow, pl.ds(ecol, 16)]
                        for t in range(16):
                            iv = ivv[t]
                            r = g * 16 + t
                            for kk in range(Dh // 16):
                                pslab[r, pl.ds(kk * 16, 16)] = (
                                    pslab[r, pl.ds(kk * 16, 16)] * iv)
                    pltpu.sync_copy(pslab, outh.at[pl.ds(row0, NCHK)])
                return carry
            lax.fori_loop(0, SLAB // NCHK, norm_chunk, 0)

        @pl.when(c == 0)
        def _():
            writeout(outl_h)

        @pl.when(c == 1)
        def _():
            writeout(outr_h)

    return sc_fn


def kernel(x, triples):
    N, D = x.shape
    NS, CH = 16, 128
    t0 = triples[:, 0]
    t2 = triples[:, 2]
    senders = jnp.concatenate([t0, t2])
    receivers = jnp.concatenate([t2, t0])
    E = senders.shape[0]

    nch = -(-E // (NS * CH))
    nch += -nch % SB
    epad = NS * nch * CH
    pad = epad - E
    senders_p = jnp.concatenate(
        [senders, jnp.zeros((pad,), jnp.int32)]).reshape(NS, nch, CH)
    receivers_p = jnp.concatenate(
        [receivers, jnp.full((pad,), N, jnp.int32)]).reshape(NS, nch, CH)

    xl = x[:, : D // 2]
    xr = x[:, D // 2:]

    sc_fn = _build_sc_call(N, D, nch)
    out_l, out_r = sc_fn(xl, xr, senders_p, receivers_p)
    return jnp.concatenate([out_l, out_r], axis=1)


# speedup vs baseline: 17.9240x; 1.0544x over previous
import functools

import jax
import jax.numpy as jnp
from jax import lax
from jax.experimental import pallas as pl
from jax.experimental.pallas import tpu as pltpu
from jax.experimental.pallas import tpu_sc as plsc

NBUF = 2
SB = 32


def _build_sc_call(N, D, NCH):
    Dh = D // 2
    NS = 16
    CH = 128
    SLAB = 640
    R_ACC = NS * SLAB
    NCHK = 80
    HR = R_ACC // CH
    XR = N // NS

    mesh = plsc.VectorSubcoreMesh(core_axis_name="c", subcore_axis_name="s")

    @functools.partial(
        pl.kernel,
        mesh=mesh,
        compiler_params=pltpu.CompilerParams(use_tc_tiling_on_sc=False,
                                             needs_layout_passes=False),
        out_type=jax.ShapeDtypeStruct((N, D), jnp.float32),
        scratch_types=[
            pltpu.VMEM((SB, CH), jnp.int32),
            pltpu.VMEM((SB, CH), jnp.int32),
            pltpu.VMEM((NBUF, CH, Dh), jnp.float32),
            pltpu.VMEM((HR, CH), jnp.float32),
            pltpu.VMEM((HR,), jnp.int32),
            pltpu.VMEM((NCHK, Dh), jnp.float32),
            pltpu.VMEM((HR // NS, CH), jnp.float32),
            pltpu.VMEM((HR // NS, CH), jnp.float32),
            pltpu.VMEM_SHARED((R_ACC, Dh), jnp.float32),
            pltpu.VMEM_SHARED((HR, CH), jnp.float32),
            pltpu.VMEM_SHARED((N, Dh), jnp.float32),
        ] + [pltpu.SemaphoreType.DMA] * (2 * NBUF),
    )
    def sc_fn(xl_h, xr_h, s_h, r_h, out_h,
              sidx, ridx, gbuf, hist, hrows, pslab, dslab, islab,
              acc, dacc, xs, *sems):
        c = lax.axis_index("c")
        s = lax.axis_index("s")
        base = s * SLAB

        def fill_ps(i, carry):
            for k in range(Dh // 16):
                pslab[i, pl.ds(k * 16, 16)] = jnp.zeros((16,), jnp.float32)
            return carry
        lax.fori_loop(0, NCHK, fill_ps, 0)

        def fill_hist(i, carry):
            for k in range(CH // 16):
                hist[i, pl.ds(k * 16, 16)] = jnp.zeros((16,), jnp.float32)
            return carry
        lax.fori_loop(0, HR, fill_hist, 0)

        for g in range(HR // 16):
            hrows[pl.ds(g * 16, 16)] = lax.iota(jnp.int32, 16) + (g * 16)

        for b in range(SLAB // NCHK):
            pltpu.sync_copy(pslab, acc.at[pl.ds(base + b * NCHK, NCHK)])

        @pl.when(s == 0)
        def _():
            pltpu.sync_copy(hist, dacc)

        def stage_x(xh):
            pltpu.sync_copy(xh.at[pl.ds(s * XR, XR)], xs.at[pl.ds(s * XR, XR)])

        @pl.when(c == 0)
        def _():
            stage_x(xl_h)

        @pl.when(c == 1)
        def _():
            stage_x(xr_h)

        plsc.subcore_barrier()

        ones16 = jnp.ones((16,), jnp.float32)

        def stage(t, carry):
            pltpu.sync_copy(s_h.at[s, pl.ds(t * SB, SB)], sidx)
            pltpu.sync_copy(r_h.at[s, pl.ds(t * SB, SB)], ridx)

            def rnd(i, carry2):
                g = i * NBUF
                handles = [
                    pltpu.async_copy(xs.at[sidx.at[g + b]], gbuf.at[b],
                                     sems[b])
                    for b in range(NBUF)
                ]
                for b in range(NBUF):
                    for k in range(CH // 16):
                        rv = ridx[g + b, pl.ds(k * 16, 16)]
                        rrow = lax.shift_right_logical(rv, 7)
                        rcol = lax.bitwise_and(rv, 127)
                        plsc.addupdate_scatter(hist, [rrow, rcol], ones16)
                shandles = []
                for b in range(NBUF):
                    handles[b].wait()
                    shandles.append(
                        pltpu.async_copy(gbuf.at[b], acc.at[ridx.at[g + b]],
                                         sems[NBUF + b], add=True))
                for h in shandles:
                    h.wait()
                return carry2
            lax.fori_loop(0, SB // NBUF, rnd, 0)
            return carry
        lax.fori_loop(0, NCH // SB, stage, 0)

        pltpu.sync_copy(hist, dacc.at[hrows], add=True)

        plsc.subcore_barrier()

        pltpu.sync_copy(dacc.at[pl.ds(s * (HR // NS), HR // NS)], dslab)

        def inv_r(r, carry):
            for k in range(CH // 16):
                d = dslab[r, pl.ds(k * 16, 16)]
                islab[r, pl.ds(k * 16, 16)] = 1.0 / jnp.maximum(d, 1.0)
            return carry
        lax.fori_loop(0, HR // NS, inv_r, 0)

        def norm_chunk(k, carry):
            row0 = base + k * NCHK

            @pl.when(row0 + NCHK <= N)
            def _():
                pltpu.sync_copy(acc.at[pl.ds(row0, NCHK)], pslab)
                for g in range(NCHK // 16):
                    e0 = k * NCHK + g * 16
                    erow = e0 // CH
                    ecol = e0 % CH
                    ivv = islab[erow, pl.ds(ecol, 16)]
                    for t in range(16):
                        iv = ivv[t]
                        r = g * 16 + t
                        for kk in range(Dh // 16):
                            pslab[r, pl.ds(kk * 16, 16)] = (
                                pslab[r, pl.ds(kk * 16, 16)] * iv)
                pltpu.sync_copy(
                    pslab, out_h.at[pl.ds(row0, NCHK), pl.ds(c * Dh, Dh)])
            return carry
        lax.fori_loop(0, SLAB // NCHK, norm_chunk, 0)

    return sc_fn


def kernel(x, triples):
    N, D = x.shape
    NS, CH = 16, 128
    t0 = triples[:, 0]
    t2 = triples[:, 2]
    senders = jnp.concatenate([t0, t2])
    receivers = jnp.concatenate([t2, t0])
    E = senders.shape[0]

    nch = -(-E // (NS * CH))
    nch += -nch % SB
    epad = NS * nch * CH
    pad = epad - E
    senders_p = jnp.concatenate(
        [senders, jnp.zeros((pad,), jnp.int32)]).reshape(NS, nch, CH)
    receivers_p = jnp.concatenate(
        [receivers, jnp.full((pad,), N, jnp.int32)]).reshape(NS, nch, CH)

    xl = x[:, : D // 2]
    xr = x[:, D // 2:]

    sc_fn = _build_sc_call(N, D, nch)
    return sc_fn(xl, xr, senders_p, receivers_p)


# speedup vs baseline: 23.7542x; 1.3253x over previous
import functools

import jax
import jax.numpy as jnp
from jax import lax
from jax.experimental import pallas as pl
from jax.experimental.pallas import tpu as pltpu
from jax.experimental.pallas import tpu_sc as plsc

SB = 32


def _build_sc_call(N, D, NCH):
    Dh = D // 2
    NS = 16
    CH = 128
    SLAB = 640
    R_ACC = NS * SLAB
    NCHK = 80
    XR = N // NS
    RPS = SB // 2

    mesh = plsc.VectorSubcoreMesh(core_axis_name="c", subcore_axis_name="s")

    @functools.partial(
        pl.kernel,
        mesh=mesh,
        compiler_params=pltpu.CompilerParams(use_tc_tiling_on_sc=False,
                                             needs_layout_passes=False),
        out_type=jax.ShapeDtypeStruct((N, D), jnp.float32),
        scratch_types=[
            pltpu.VMEM((SB, CH), jnp.int32),
            pltpu.VMEM((SB, CH), jnp.int32),
            pltpu.VMEM((4, CH, Dh), jnp.float32),
            pltpu.VMEM((CH,), jnp.float32),
            pltpu.VMEM((SLAB,), jnp.float32),
            pltpu.VMEM((NCHK, Dh), jnp.float32),
            pltpu.VMEM((SLAB,), jnp.float32),
            pltpu.VMEM((SLAB,), jnp.float32),
            pltpu.VMEM_SHARED((R_ACC, Dh), jnp.float32),
            pltpu.VMEM_SHARED((R_ACC,), jnp.float32),
            pltpu.VMEM_SHARED((N, Dh), jnp.float32),
        ] + [pltpu.SemaphoreType.DMA] * 8,
    )
    def sc_fn(x_h, s_h, r_h, out_h,
              sidx, ridx, gbuf, ones, zvec, pslab, dslab, islab,
              acc, dacc, xs, *sems):
        c = lax.axis_index("c")
        s = lax.axis_index("s")
        base = s * SLAB

        def fill_ps(i, carry):
            for k in range(Dh // 16):
                pslab[i, pl.ds(k * 16, 16)] = jnp.zeros((16,), jnp.float32)
            return carry
        lax.fori_loop(0, NCHK, fill_ps, 0)

        def fill_zv(i, carry):
            zvec[pl.ds(i * 16, 16)] = jnp.zeros((16,), jnp.float32)
            return carry
        lax.fori_loop(0, SLAB // 16, fill_zv, 0)

        def fill_on(i, carry):
            ones[pl.ds(i * 16, 16)] = jnp.ones((16,), jnp.float32)
            return carry
        lax.fori_loop(0, CH // 16, fill_on, 0)

        for b in range(SLAB // NCHK):
            pltpu.sync_copy(pslab, acc.at[pl.ds(base + b * NCHK, NCHK)])
        pltpu.sync_copy(zvec, dacc.at[pl.ds(base, SLAB)])

        pltpu.sync_copy(x_h.at[pl.ds(s * XR, XR), pl.ds(c * Dh, Dh)],
                        xs.at[pl.ds(s * XR, XR)])

        plsc.subcore_barrier()

        def issue_gathers(r2, grp):
            for half in range(2):
                b = grp * 2 + half
                pltpu.async_copy(xs.at[sidx.at[r2 + half]], gbuf.at[b],
                                 sems[b])

        def wait_gathers(r2, grp):
            for half in range(2):
                b = grp * 2 + half
                pltpu.make_async_copy(xs.at[sidx.at[r2 + half]], gbuf.at[b],
                                      sems[b]).wait()

        def issue_scatters(r2, grp):
            for half in range(2):
                b = grp * 2 + half
                pltpu.async_copy(gbuf.at[b], acc.at[ridx.at[r2 + half]],
                                 sems[4 + b], add=True)

        def drain_scatters(grp):
            for half in range(2):
                b = grp * 2 + half
                pltpu.make_async_copy(gbuf.at[b], acc.at[ridx.at[half]],
                                      sems[4 + b]).wait()

        def deg_scatters(r2):
            for half in range(2):
                pltpu.sync_copy(ones, dacc.at[ridx.at[r2 + half]], add=True)

        def stage(t, carry):
            pltpu.sync_copy(s_h.at[s, pl.ds(t * SB, SB)], sidx)
            pltpu.sync_copy(r_h.at[s, pl.ds(t * SB, SB)], ridx)
            issue_gathers(0, 0)

            def body(j, carry2):
                r2 = 4 * j

                @pl.when(j > 0)
                def _():
                    drain_scatters(1)
                @pl.when(r2 + 2 < SB)
                def _():
                    issue_gathers(r2 + 2, 1)
                wait_gathers(r2, 0)
                issue_scatters(r2, 0)
                deg_scatters(r2)

                drain_scatters(0)

                @pl.when(r2 + 4 < SB)
                def _():
                    issue_gathers(r2 + 4, 0)
                wait_gathers(r2 + 2, 1)
                issue_scatters(r2 + 2, 1)
                deg_scatters(r2 + 2)
                return carry2
            lax.fori_loop(0, RPS // 2, body, 0)
            drain_scatters(1)
            return carry
        lax.fori_loop(0, NCH // SB, stage, 0)

        plsc.subcore_barrier()

        pltpu.sync_copy(dacc.at[pl.ds(base, SLAB)], dslab)

        def inv_f(i, carry):
            d = dslab[pl.ds(i * 16, 16)]
            islab[pl.ds(i * 16, 16)] = 1.0 / jnp.maximum(d, 1.0)
            return carry
        lax.fori_loop(0, SLAB // 16, inv_f, 0)

        def norm_chunk(k, carry):
            row0 = base + k * NCHK

            @pl.when(row0 + NCHK <= N)
            def _():
                pltpu.sync_copy(acc.at[pl.ds(row0, NCHK)], pslab)
                for g in range(NCHK // 16):
                    ivv = islab[pl.ds(k * NCHK + g * 16, 16)]
                    for t in range(16):
                        iv = ivv[t]
                        r = g * 16 + t
                        for kk in range(Dh // 16):
                            pslab[r, pl.ds(kk * 16, 16)] = (
                                pslab[r, pl.ds(kk * 16, 16)] * iv)
                pltpu.sync_copy(
                    pslab, out_h.at[pl.ds(row0, NCHK), pl.ds(c * Dh, Dh)])
            return carry
        lax.fori_loop(0, SLAB // NCHK, norm_chunk, 0)

    return sc_fn


def kernel(x, triples):
    N, D = x.shape
    NS, CH = 16, 128
    t0 = triples[:, 0]
    t2 = triples[:, 2]
    senders = jnp.concatenate([t0, t2])
    receivers = jnp.concatenate([t2, t0])
    E = senders.shape[0]

    nch = -(-E // (NS * CH))
    nch += -nch % SB
    epad = NS * nch * CH
    pad = epad - E
    senders_p = jnp.concatenate(
        [senders, jnp.zeros((pad,), jnp.int32)]).reshape(NS, nch, CH)
    receivers_p = jnp.concatenate(
        [receivers, jnp.full((pad,), N, jnp.int32)]).reshape(NS, nch, CH)

    sc_fn = _build_sc_call(N, D, nch)
    return sc_fn(x, senders_p, receivers_p)


# speedup vs baseline: 24.1478x; 1.0166x over previous
import functools

import jax
import jax.numpy as jnp
from jax import lax
from jax.experimental import pallas as pl
from jax.experimental.pallas import tpu as pltpu
from jax.experimental.pallas import tpu_sc as plsc

SB = 32


def _build_sc_call(N, D, NCH):
    Dh = D // 2
    NS = 16
    CH = 128
    SLAB = 640
    R_ACC = NS * SLAB
    NCHK = 80
    XR = N // NS
    RPS = SB // 2

    mesh = plsc.VectorSubcoreMesh(core_axis_name="c", subcore_axis_name="s")

    @functools.partial(
        pl.kernel,
        mesh=mesh,
        compiler_params=pltpu.CompilerParams(use_tc_tiling_on_sc=False,
                                             needs_layout_passes=False),
        out_type=jax.ShapeDtypeStruct((N, D), jnp.float32),
        scratch_types=[
            pltpu.VMEM((SB, CH), jnp.int32),
            pltpu.VMEM((SB, CH), jnp.int32),
            pltpu.VMEM((4, CH, Dh), jnp.float32),
            pltpu.VMEM((CH,), jnp.float32),
            pltpu.VMEM((SLAB,), jnp.float32),
            pltpu.VMEM((NCHK, Dh), jnp.float32),
            pltpu.VMEM((SLAB,), jnp.float32),
            pltpu.VMEM((SLAB,), jnp.float32),
            pltpu.VMEM_SHARED((R_ACC, Dh), jnp.float32),
            pltpu.VMEM_SHARED((R_ACC,), jnp.float32),
            pltpu.VMEM_SHARED((N, Dh), jnp.float32),
        ] + [pltpu.SemaphoreType.DMA] * 10,
    )
    def sc_fn(x_h, s_h, r_h, out_h,
              sidx, ridx, gbuf, ones, zvec, pslab, dslab, islab,
              acc, dacc, xs, *sems):
        c = lax.axis_index("c")
        s = lax.axis_index("s")
        base = s * SLAB

        def fill_ps(i, carry):
            for k in range(Dh // 16):
                pslab[i, pl.ds(k * 16, 16)] = jnp.zeros((16,), jnp.float32)
            return carry
        lax.fori_loop(0, NCHK, fill_ps, 0)

        def fill_zv(i, carry):
            zvec[pl.ds(i * 16, 16)] = jnp.zeros((16,), jnp.float32)
            return carry
        lax.fori_loop(0, SLAB // 16, fill_zv, 0)

        def fill_on(i, carry):
            ones[pl.ds(i * 16, 16)] = jnp.ones((16,), jnp.float32)
            return carry
        lax.fori_loop(0, CH // 16, fill_on, 0)

        for b in range(SLAB // NCHK):
            pltpu.sync_copy(pslab, acc.at[pl.ds(base + b * NCHK, NCHK)])
        pltpu.sync_copy(zvec, dacc.at[pl.ds(base, SLAB)])

        pltpu.sync_copy(x_h.at[pl.ds(s * XR, XR), pl.ds(c * Dh, Dh)],
                        xs.at[pl.ds(s * XR, XR)])

        plsc.subcore_barrier()

        def issue_gathers(r2, grp):
            for half in range(2):
                b = grp * 2 + half
                pltpu.async_copy(xs.at[sidx.at[r2 + half]], gbuf.at[b],
                                 sems[b])

        def wait_gathers(r2, grp):
            for half in range(2):
                b = grp * 2 + half
                pltpu.make_async_copy(xs.at[sidx.at[r2 + half]], gbuf.at[b],
                                      sems[b]).wait()

        def issue_scatters(r2, grp):
            for half in range(2):
                b = grp * 2 + half
                pltpu.async_copy(gbuf.at[b], acc.at[ridx.at[r2 + half]],
                                 sems[4 + b], add=True)

        def drain_scatters(grp):
            for half in range(2):
                b = grp * 2 + half
                pltpu.make_async_copy(gbuf.at[b], acc.at[ridx.at[half]],
                                      sems[4 + b]).wait()
                pltpu.make_async_copy(ones, dacc.at[ridx.at[half]],
                                      sems[8 + grp]).wait()

        def deg_scatters(r2, grp):
            for half in range(2):
                pltpu.async_copy(ones, dacc.at[ridx.at[r2 + half]],
                                 sems[8 + grp], add=True)

        def stage(t, carry):
            pltpu.sync_copy(s_h.at[s, pl.ds(t * SB, SB)], sidx)
            pltpu.sync_copy(r_h.at[s, pl.ds(t * SB, SB)], ridx)
            issue_gathers(0, 0)

            def body(j, carry2):
                r2 = 4 * j

                @pl.when(j > 0)
                def _():
                    drain_scatters(1)
                @pl.when(r2 + 2 < SB)
                def _():
                    issue_gathers(r2 + 2, 1)
                wait_gathers(r2, 0)
                issue_scatters(r2, 0)
                deg_scatters(r2, 0)

                drain_scatters(0)

                @pl.when(r2 + 4 < SB)
                def _():
                    issue_gathers(r2 + 4, 0)
                wait_gathers(r2 + 2, 1)
                issue_scatters(r2 + 2, 1)
                deg_scatters(r2 + 2, 1)
                return carry2
            lax.fori_loop(0, RPS // 2, body, 0)
            drain_scatters(1)
            return carry
        lax.fori_loop(0, NCH // SB, stage, 0)

        plsc.subcore_barrier()

        pltpu.sync_copy(dacc.at[pl.ds(base, SLAB)], dslab)

        def inv_f(i, carry):
            d = dslab[pl.ds(i * 16, 16)]
            islab[pl.ds(i * 16, 16)] = 1.0 / jnp.maximum(d, 1.0)
            return carry
        lax.fori_loop(0, SLAB // 16, inv_f, 0)

        def norm_chunk(k, carry):
            row0 = base + k * NCHK

            @pl.when(row0 + NCHK <= N)
            def _():
                pltpu.sync_copy(acc.at[pl.ds(row0, NCHK)], pslab)
                for g in range(NCHK // 16):
                    ivv = islab[pl.ds(k * NCHK + g * 16, 16)]
                    for t in range(16):
                        iv = ivv[t]
                        r = g * 16 + t
                        for kk in range(Dh // 16):
                            pslab[r, pl.ds(kk * 16, 16)] = (
                                pslab[r, pl.ds(kk * 16, 16)] * iv)
                pltpu.sync_copy(
                    pslab, out_h.at[pl.ds(row0, NCHK), pl.ds(c * Dh, Dh)])
            return carry
        lax.fori_loop(0, SLAB // NCHK, norm_chunk, 0)

    return sc_fn


def kernel(x, triples):
    N, D = x.shape
    NS, CH = 16, 128
    t0 = triples[:, 0]
    t2 = triples[:, 2]
    senders = jnp.concatenate([t0, t2])
    receivers = jnp.concatenate([t2, t0])
    E = senders.shape[0]

    nch = -(-E // (NS * CH))
    nch += -nch % SB
    epad = NS * nch * CH
    pad = epad - E
    senders_p = jnp.concatenate(
        [senders, jnp.zeros((pad,), jnp.int32)]).reshape(NS, nch, CH)
    receivers_p = jnp.concatenate(
        [receivers, jnp.full((pad,), N, jnp.int32)]).reshape(NS, nch, CH)

    sc_fn = _build_sc_call(N, D, nch)
    return sc_fn(x, senders_p, receivers_p)


# speedup vs baseline: 24.5460x; 1.0165x over previous
import functools

import jax
import jax.numpy as jnp
from jax import lax
from jax.experimental import pallas as pl
from jax.experimental.pallas import tpu as pltpu
from jax.experimental.pallas import tpu_sc as plsc

SB = 32


def _build_sc_call(N, D, NCH):
    Dh = D // 2
    NS = 16
    CH = 128
    SLAB = 640
    R_ACC = NS * SLAB
    NCHK = 80
    XR = N // NS
    RPS = SB // 2

    mesh = plsc.VectorSubcoreMesh(core_axis_name="c", subcore_axis_name="s")

    @functools.partial(
        pl.kernel,
        mesh=mesh,
        compiler_params=pltpu.CompilerParams(use_tc_tiling_on_sc=False,
                                             needs_layout_passes=False),
        out_type=jax.ShapeDtypeStruct((N, D), jnp.float32),
        scratch_types=[
            pltpu.VMEM((SB, CH), jnp.int32),
            pltpu.VMEM((SB, CH), jnp.int32),
            pltpu.VMEM((4, CH, Dh), jnp.float32),
            pltpu.VMEM((CH,), jnp.float32),
            pltpu.VMEM((SLAB,), jnp.float32),
            pltpu.VMEM((NCHK, Dh), jnp.float32),
            pltpu.VMEM((SLAB,), jnp.float32),
            pltpu.VMEM((SLAB,), jnp.float32),
            pltpu.VMEM_SHARED((R_ACC, Dh), jnp.float32),
            pltpu.VMEM_SHARED((R_ACC,), jnp.float32),
            pltpu.VMEM_SHARED((N, Dh), jnp.float32),
        ] + [pltpu.SemaphoreType.DMA] * 10,
    )
    def sc_fn(x_h, s_h, r_h, out_h,
              sidx, ridx, gbuf, ones, zvec, pslab, dslab, islab,
              acc, dacc, xs, *sems):
        c = lax.axis_index("c")
        s = lax.axis_index("s")
        base = s * SLAB

        def fill_ps(i, carry):
            for k in range(Dh // 16):
                pslab[i, pl.ds(k * 16, 16)] = jnp.zeros((16,), jnp.float32)
            return carry
        lax.fori_loop(0, NCHK, fill_ps, 0)

        def fill_zv(i, carry):
            zvec[pl.ds(i * 16, 16)] = jnp.zeros((16,), jnp.float32)
            return carry
        lax.fori_loop(0, SLAB // 16, fill_zv, 0)

        def fill_on(i, carry):
            ones[pl.ds(i * 16, 16)] = jnp.ones((16,), jnp.float32)
            return carry
        lax.fori_loop(0, CH // 16, fill_on, 0)

        for b in range(SLAB // NCHK):
            pltpu.async_copy(pslab, acc.at[pl.ds(base + b * NCHK, NCHK)],
                             sems[0])
        pltpu.async_copy(zvec, dacc.at[pl.ds(base, SLAB)], sems[1])
        pltpu.async_copy(x_h.at[pl.ds(s * XR, XR), pl.ds(c * Dh, Dh)],
                         xs.at[pl.ds(s * XR, XR)], sems[2])
        for b in range(SLAB // NCHK):
            pltpu.make_async_copy(pslab, acc.at[pl.ds(base + b * NCHK, NCHK)],
                                  sems[0]).wait()
        pltpu.make_async_copy(zvec, dacc.at[pl.ds(base, SLAB)], sems[1]).wait()
        pltpu.make_async_copy(x_h.at[pl.ds(s * XR, XR), pl.ds(c * Dh, Dh)],
                              xs.at[pl.ds(s * XR, XR)], sems[2]).wait()

        plsc.subcore_barrier()

        def issue_gathers(r2, grp):
            for half in range(2):
                b = grp * 2 + half
                pltpu.async_copy(xs.at[sidx.at[r2 + half]], gbuf.at[b],
                                 sems[b])

        def wait_gathers(r2, grp):
            for half in range(2):
                b = grp * 2 + half
                pltpu.make_async_copy(xs.at[sidx.at[r2 + half]], gbuf.at[b],
                                      sems[b]).wait()

        def issue_scatters(r2, grp):
            for half in range(2):
                b = grp * 2 + half
                pltpu.async_copy(gbuf.at[b], acc.at[ridx.at[r2 + half]],
                                 sems[4 + b], add=True)

        def drain_scatters(grp):
            for half in range(2):
                b = grp * 2 + half
                pltpu.make_async_copy(gbuf.at[b], acc.at[ridx.at[half]],
                                      sems[4 + b]).wait()
                pltpu.make_async_copy(ones, dacc.at[ridx.at[half]],
                                      sems[8 + grp]).wait()

        def deg_scatters(r2, grp):
            for half in range(2):
                pltpu.async_copy(ones, dacc.at[ridx.at[r2 + half]],
                                 sems[8 + grp], add=True)

        def stage(t, carry):
            pltpu.sync_copy(s_h.at[s, pl.ds(t * SB, SB)], sidx)
            pltpu.sync_copy(r_h.at[s, pl.ds(t * SB, SB)], ridx)
            issue_gathers(0, 0)

            def body(j, carry2):
                r2 = 4 * j

                @pl.when(j > 0)
                def _():
                    drain_scatters(1)
                @pl.when(r2 + 2 < SB)
                def _():
                    issue_gathers(r2 + 2, 1)
                wait_gathers(r2, 0)
                issue_scatters(r2, 0)
                deg_scatters(r2, 0)

                drain_scatters(0)

                @pl.when(r2 + 4 < SB)
                def _():
                    issue_gathers(r2 + 4, 0)
                wait_gathers(r2 + 2, 1)
                issue_scatters(r2 + 2, 1)
                deg_scatters(r2 + 2, 1)
                return carry2
            lax.fori_loop(0, RPS // 2, body, 0)
            drain_scatters(1)
            return carry
        lax.fori_loop(0, NCH // SB, stage, 0)

        plsc.subcore_barrier()

        pltpu.sync_copy(dacc.at[pl.ds(base, SLAB)], dslab)

        def inv_f(i, carry):
            d = dslab[pl.ds(i * 16, 16)]
            islab[pl.ds(i * 16, 16)] = 1.0 / jnp.maximum(d, 1.0)
            return carry
        lax.fori_loop(0, SLAB // 16, inv_f, 0)

        def norm_chunk(k, carry):
            row0 = base + k * NCHK

            @pl.when(row0 + NCHK <= N)
            def _():
                pltpu.sync_copy(acc.at[pl.ds(row0, NCHK)], pslab)
                for g in range(NCHK // 16):
                    ivv = islab[pl.ds(k * NCHK + g * 16, 16)]
                    for t in range(16):
                        iv = ivv[t]
                        r = g * 16 + t
                        for kk in range(Dh // 16):
                            pslab[r, pl.ds(kk * 16, 16)] = (
                                pslab[r, pl.ds(kk * 16, 16)] * iv)
                pltpu.sync_copy(
                    pslab, out_h.at[pl.ds(row0, NCHK), pl.ds(c * Dh, Dh)])
            return carry
        lax.fori_loop(0, SLAB // NCHK, norm_chunk, 0)

    return sc_fn


def kernel(x, triples):
    N, D = x.shape
    NS, CH = 16, 128
    t0 = triples[:, 0]
    t2 = triples[:, 2]
    senders = jnp.concatenate([t0, t2])
    receivers = jnp.concatenate([t2, t0])
    E = senders.shape[0]

    nch = -(-E // (NS * CH))
    nch += -nch % SB
    epad = NS * nch * CH
    pad = epad - E
    senders_p = jnp.concatenate(
        [senders, jnp.zeros((pad,), jnp.int32)]).reshape(NS, nch, CH)
    receivers_p = jnp.concatenate(
        [receivers, jnp.full((pad,), N, jnp.int32)]).reshape(NS, nch, CH)

    sc_fn = _build_sc_call(N, D, nch)
    return sc_fn(x, senders_p, receivers_p)
